# Initial kernel scaffold; baseline (speedup 1.0000x reference)
import functools

import jax
import jax.numpy as jnp
from jax import lax
from jax.experimental import pallas as pl
from jax.experimental.pallas import tpu as pltpu
from jax.experimental.pallas import tpu_sc as plsc

N = 10000
E = 320000
NP = 10240
NC = 2
NS = 16
NW = NC * NS
EPW = E // NW
CH = 80
NCHUNK = EPW // CH
ROWS_PER_TILE = NP // NS

_mesh = plsc.VectorSubcoreMesh(core_axis_name="c", subcore_axis_name="s")


@functools.partial(
    pl.kernel,
    mesh=_mesh,
    out_type=jax.ShapeDtypeStruct((NC, 2, 80, 128), jnp.float32),
    scratch_types=[
        pltpu.VMEM((NCHUNK, CH), jnp.int32),
        pltpu.VMEM((NCHUNK, CH), jnp.int32),
        pltpu.VMEM((80, 128), jnp.float32),
        pltpu.VMEM((80, 128), jnp.float32),
        pltpu.VMEM((80,), jnp.int32),
        pltpu.VMEM_SHARED((80, 128), jnp.float32),
        pltpu.VMEM_SHARED((80, 128), jnp.float32),
    ],
)
def _deg_kernel(src_hbm, dst_hbm, out_hbm, sv, dv, dego, degi, rowid, sh_o, sh_i):
    c = lax.axis_index("c")
    s = lax.axis_index("s")
    w = c * NS + s

    pltpu.sync_copy(src_hbm.at[pl.ds(w * NCHUNK, NCHUNK)], sv)
    pltpu.sync_copy(dst_hbm.at[pl.ds(w * NCHUNK, NCHUNK)], dv)

    def zrow(r, carry):
        for k in range(8):
            dego[r, pl.ds(k * 16, 16)] = jnp.zeros((16,), jnp.float32)
            degi[r, pl.ds(k * 16, 16)] = jnp.zeros((16,), jnp.float32)
        return carry
    lax.fori_loop(0, 80, zrow, 0)
    for k in range(5):
        rowid[pl.ds(k * 16, 16)] = lax.iota(jnp.int32, 16) + k * 16

    ones = jnp.ones((16,), jnp.float32)

    def body(r, carry):
        for k in range(CH // 16):
            si = sv[r, pl.ds(k * 16, 16)]
            di = dv[r, pl.ds(k * 16, 16)]
            plsc.addupdate_scatter(
                dego, [lax.shift_right_logical(si, 7),
                       lax.bitwise_and(si, 127)], ones)
            plsc.addupdate_scatter(
                degi, [lax.shift_right_logical(di, 7),
                       lax.bitwise_and(di, 127)], ones)
        return carry
    lax.fori_loop(0, NCHUNK, body, 0)

    @pl.when(s == 0)
    def _():
        pltpu.sync_copy(dego, sh_o.at[rowid])
        pltpu.sync_copy(degi, sh_i.at[rowid])
    plsc.subcore_barrier()

    @pl.when(s != 0)
    def _():
        pltpu.sync_copy(dego, sh_o.at[rowid], add=True)
        pltpu.sync_copy(degi, sh_i.at[rowid], add=True)
    plsc.subcore_barrier()

    @pl.when(s == 0)
    def _():
        pltpu.sync_copy(sh_o, out_hbm.at[c, 0])
        pltpu.sync_copy(sh_i, out_hbm.at[c, 1])


@functools.partial(
    pl.kernel,
    mesh=_mesh,
    out_type=jax.ShapeDtypeStruct((NC, NP, 128), jnp.float32),
    scratch_types=[
        pltpu.VMEM((NCHUNK, CH), jnp.int32),
        pltpu.VMEM((NCHUNK, CH), jnp.int32),
        pltpu.VMEM((2, CH, 128), jnp.float32),
        pltpu.VMEM((128, 128), jnp.float32),
        pltpu.VMEM_SHARED((NP, 128), jnp.float32),
        pltpu.SemaphoreType.DMA,
    ],
)
def _agg_kernel(h_hbm, src_hbm, dst_hbm, out_hbm, sidx, didx, rows, zbuf, acc, gsem):
    c = lax.axis_index("c")
    s = lax.axis_index("s")
    w = c * NS + s

    def zrow(r, carry):
        for k in range(8):
            zbuf[r, pl.ds(k * 16, 16)] = jnp.zeros((16,), jnp.float32)
        return carry
    lax.fori_loop(0, 128, zrow, 0)
    for t in range(ROWS_PER_TILE // 128):
        pltpu.sync_copy(zbuf, acc.at[pl.ds(s * ROWS_PER_TILE + t * 128, 128)])

    pltpu.sync_copy(src_hbm.at[pl.ds(w * NCHUNK, NCHUNK)], sidx)
    pltpu.sync_copy(dst_hbm.at[pl.ds(w * NCHUNK, NCHUNK)], didx)

    plsc.subcore_barrier()

    pltpu.async_copy(h_hbm.at[sidx.at[0]], rows.at[0], gsem)

    def body(j, carry):
        b = lax.bitwise_and(j, 1)
        pltpu.make_async_copy(h_hbm.at[sidx.at[j]], rows.at[b], gsem).wait()
        pltpu.async_copy(h_hbm.at[sidx.at[j + 1]], rows.at[1 - b], gsem)
        pltpu.sync_copy(rows.at[b], acc.at[didx.at[j]], add=True)
        return carry
    lax.fori_loop(0, NCHUNK - 1, body, 0)

    last = NCHUNK - 1
    bl = last % 2
    pltpu.make_async_copy(h_hbm.at[sidx.at[last]], rows.at[bl], gsem).wait()
    pltpu.sync_copy(rows.at[bl], acc.at[didx.at[last]], add=True)

    plsc.subcore_barrier()
    pltpu.sync_copy(acc.at[pl.ds(s * ROWS_PER_TILE, ROWS_PER_TILE)],
                    out_hbm.at[c, pl.ds(s * ROWS_PER_TILE, ROWS_PER_TILE)])


_R = 2560
_G = NP // _R


def _mm1_body(f_ref, w_ref, n_ref, o_ref):
    o_ref[...] = jnp.dot(f_ref[...], w_ref[...],
                         preferred_element_type=jnp.float32) * n_ref[...]


def _mm1(fpad, W1, nsrc):
    return pl.pallas_call(
        _mm1_body,
        grid=(_G,),
        in_specs=[
            pl.BlockSpec((_R, 128), lambda i: (i, 0)),
            pl.BlockSpec((128, 128), lambda i: (0, 0)),
            pl.BlockSpec((_R, 1), lambda i: (i, 0)),
        ],
        out_specs=pl.BlockSpec((_R, 128), lambda i: (i, 0)),
        out_shape=jax.ShapeDtypeStruct((NP, 128), jnp.float32),
    )(fpad, W1, nsrc)


def _mm2_body(p_ref, nd_ref, b_ref, w_ref, ns_ref, o_ref):
    x = (p_ref[0] + p_ref[1]) * nd_ref[...] + b_ref[...]
    x = jnp.maximum(x, 0.0)
    o_ref[...] = jnp.dot(x, w_ref[...],
                         preferred_element_type=jnp.float32) * ns_ref[...]


def _mm2(parts, ndst, b1r, W2, nsrc):
    return pl.pallas_call(
        _mm2_body,
        grid=(_G,),
        in_specs=[
            pl.BlockSpec((NC, _R, 128), lambda i: (0, i, 0)),
            pl.BlockSpec((_R, 1), lambda i: (i, 0)),
            pl.BlockSpec((1, 128), lambda i: (0, 0)),
            pl.BlockSpec((128, 128), lambda i: (0, 0)),
            pl.BlockSpec((_R, 1), lambda i: (i, 0)),
        ],
        out_specs=pl.BlockSpec((_R, 128), lambda i: (i, 0)),
        out_shape=jax.ShapeDtypeStruct((NP, 128), jnp.float32),
    )(parts, ndst, b1r, W2, nsrc)


def _final_body(p_ref, nd_ref, b_ref, w_ref, bp_ref, o_ref):
    x = (p_ref[0] + p_ref[1]) * nd_ref[...] + b_ref[...]
    x = jnp.maximum(x, 0.0)
    lg = jnp.dot(x, w_ref[...], preferred_element_type=jnp.float32) + bp_ref[...]
    m = jnp.max(lg, axis=1, keepdims=True)
    e = jnp.exp(lg - m)
    o_ref[...] = e / jnp.sum(e, axis=1, keepdims=True)


def _final(parts, ndst, b2r, Wpp, bpp):
    return pl.pallas_call(
        _final_body,
        grid=(_G,),
        in_specs=[
            pl.BlockSpec((NC, _R, 128), lambda i: (0, i, 0)),
            pl.BlockSpec((_R, 1), lambda i: (i, 0)),
            pl.BlockSpec((1, 128), lambda i: (0, 0)),
            pl.BlockSpec((128, 128), lambda i: (0, 0)),
            pl.BlockSpec((1, 128), lambda i: (0, 0)),
        ],
        out_specs=pl.BlockSpec((_R, 128), lambda i: (i, 0)),
        out_shape=jax.ShapeDtypeStruct((NP, 128), jnp.float32),
    )(parts, ndst, b2r, Wpp, bpp)


def kernel(features, edge_index, edge_types, W1, b1, W2, b2, Wp, bp):
    L = Wp.shape[1]
    src2d = edge_index[0].astype(jnp.int32).reshape(E // CH, CH)
    dst2d = edge_index[1].astype(jnp.int32).reshape(E // CH, CH)

    deg_parts = _deg_kernel(src2d, dst2d)
    deg = deg_parts.sum(axis=0).reshape(2, NP)
    nsrc = lax.rsqrt(jnp.maximum(deg[0], 1.0)).reshape(NP, 1)
    ndst = lax.rsqrt(jnp.maximum(deg[1], 1.0)).reshape(NP, 1)

    fpad = jnp.pad(features, ((0, NP - N), (0, 0)))
    b1r = b1.reshape(1, 128)
    b2r = b2.reshape(1, 128)
    Wpp = jnp.pad(Wp, ((0, 0), (0, 128 - L)))
    bpp = jnp.pad(bp, (0, 128 - L), constant_values=-1e30).reshape(1, 128)

    h1 = _mm1(fpad, W1, nsrc)
    p1 = _agg_kernel(h1, src2d, dst2d)
    h2 = _mm2(p1, ndst, b1r, W2, nsrc)
    p2 = _agg_kernel(p2_in := h2, src2d, dst2d)
    out = _final(p2, ndst, b2r, Wpp, bpp)
    return out[:N, :L]


# speedup vs baseline: 7.1892x; 7.1892x over previous
import functools

import jax
import jax.numpy as jnp
from jax import lax
from jax.experimental import pallas as pl
from jax.experimental.pallas import tpu as pltpu
from jax.experimental.pallas import tpu_sc as plsc

N = 10000
E = 320000
NP = 10240
NC = 2
NS = 16
NW = NC * NS
EPW = E // NW
CH = 80
NCHUNK = EPW // CH
ROWS_PER_TILE = NP // NS

_mesh = plsc.VectorSubcoreMesh(core_axis_name="c", subcore_axis_name="s")


STRIPE = NP // NS
DCHUNKS = E // CH // NS


@functools.partial(
    pl.kernel,
    mesh=_mesh,
    out_type=jax.ShapeDtypeStruct((NC, NP, 128), jnp.float32),
    scratch_types=[
        pltpu.VMEM((2, CH), jnp.int32),
        pltpu.VMEM((CH, 128), jnp.float32),
        pltpu.VMEM_SHARED((NP, 128), jnp.float32),
        pltpu.SemaphoreType.DMA,
    ],
)
def _deg_kernel(ei2_hbm, out_hbm, idxb, ob, acc, isem):
    c = lax.axis_index("c")
    s = lax.axis_index("s")

    def zrow(r, carry):
        for k in range(8):
            ob[r, pl.ds(k * 16, 16)] = jnp.zeros((16,), jnp.float32)
        return carry
    lax.fori_loop(0, CH, zrow, 0)
    for t in range(STRIPE // CH):
        pltpu.sync_copy(ob, acc.at[pl.ds(s * STRIPE + t * CH, CH)])

    def orow(r, carry):
        for k in range(8):
            ob[r, pl.ds(k * 16, 16)] = jnp.ones((16,), jnp.float32)
        return carry
    lax.fori_loop(0, CH, orow, 0)
    plsc.subcore_barrier()

    base = s * DCHUNKS
    pltpu.async_copy(ei2_hbm.at[c, base], idxb.at[0], isem)

    def body(j, carry):
        b = lax.bitwise_and(j, 1)
        pltpu.make_async_copy(ei2_hbm.at[c, base + j], idxb.at[b], isem).wait()

        @pl.when(j + 1 < DCHUNKS)
        def _():
            pltpu.async_copy(ei2_hbm.at[c, base + j + 1], idxb.at[1 - b], isem)

        pltpu.sync_copy(ob, acc.at[idxb.at[b]], add=True)
        return carry
    lax.fori_loop(0, DCHUNKS, body, 0)

    plsc.subcore_barrier()
    pltpu.sync_copy(acc.at[pl.ds(s * STRIPE, STRIPE)],
                    out_hbm.at[c, pl.ds(s * STRIPE, STRIPE)])


@functools.partial(
    pl.kernel,
    mesh=_mesh,
    out_type=jax.ShapeDtypeStruct((NC, NP, 128), jnp.float32),
    scratch_types=[
        pltpu.VMEM((2, 2, CH), jnp.int32),
        pltpu.VMEM((2, CH, 128), jnp.float32),
        pltpu.VMEM_SHARED((NP, 128), jnp.float32),
        pltpu.SemaphoreType.DMA,
        pltpu.SemaphoreType.DMA,
    ],
)
def _agg_kernel(h_hbm, e_hbm, out_hbm, idxb, rows, acc, isem, gsem):
    c = lax.axis_index("c")
    s = lax.axis_index("s")
    w = c * NS + s

    def zrow(r, carry):
        for k in range(8):
            rows[0, r, pl.ds(k * 16, 16)] = jnp.zeros((16,), jnp.float32)
        return carry
    lax.fori_loop(0, CH, zrow, 0)
    for t in range(STRIPE // CH):
        pltpu.sync_copy(rows.at[0], acc.at[pl.ds(s * STRIPE + t * CH, CH)])

    plsc.subcore_barrier()

    pltpu.async_copy(e_hbm.at[w, 0], idxb.at[0], isem)
    pltpu.make_async_copy(e_hbm.at[w, 0], idxb.at[0], isem).wait()
    pltpu.async_copy(e_hbm.at[w, 1], idxb.at[1], isem)
    pltpu.async_copy(h_hbm.at[idxb.at[0, 0]], rows.at[0], gsem)

    def body(j, carry):
        b = lax.bitwise_and(j, 1)
        pltpu.make_async_copy(h_hbm.at[idxb.at[b, 0]], rows.at[b], gsem).wait()

        @pl.when(j + 1 < NCHUNK)
        def _():
            pltpu.make_async_copy(e_hbm.at[w, j + 1], idxb.at[1 - b], isem).wait()
            pltpu.async_copy(h_hbm.at[idxb.at[1 - b, 0]], rows.at[1 - b], gsem)

        pltpu.sync_copy(rows.at[b], acc.at[idxb.at[b, 1]], add=True)

        @pl.when(j + 2 < NCHUNK)
        def _():
            pltpu.async_copy(e_hbm.at[w, j + 2], idxb.at[b], isem)
        return carry
    lax.fori_loop(0, NCHUNK, body, 0)

    plsc.subcore_barrier()
    pltpu.sync_copy(acc.at[pl.ds(s * STRIPE, STRIPE)],
                    out_hbm.at[c, pl.ds(s * STRIPE, STRIPE)])


_R = 2560
_G = NP // _R


def _mm1_body(f_ref, w_ref, n_ref, o_ref):
    o_ref[...] = jnp.dot(f_ref[...], w_ref[...],
                         preferred_element_type=jnp.float32) * n_ref[...]


def _mm1(fpad, W1, nsrc):
    return pl.pallas_call(
        _mm1_body,
        grid=(_G,),
        in_specs=[
            pl.BlockSpec((_R, 128), lambda i: (i, 0)),
            pl.BlockSpec((128, 128), lambda i: (0, 0)),
            pl.BlockSpec((_R, 1), lambda i: (i, 0)),
        ],
        out_specs=pl.BlockSpec((_R, 128), lambda i: (i, 0)),
        out_shape=jax.ShapeDtypeStruct((NP, 128), jnp.float32),
    )(fpad, W1, nsrc)


def _mm2_body(p_ref, nd_ref, b_ref, w_ref, ns_ref, o_ref):
    x = (p_ref[0] + p_ref[1]) * nd_ref[...] + b_ref[...]
    x = jnp.maximum(x, 0.0)
    o_ref[...] = jnp.dot(x, w_ref[...],
                         preferred_element_type=jnp.float32) * ns_ref[...]


def _mm2(parts, ndst, b1r, W2, nsrc):
    return pl.pallas_call(
        _mm2_body,
        grid=(_G,),
        in_specs=[
            pl.BlockSpec((NC, _R, 128), lambda i: (0, i, 0)),
            pl.BlockSpec((_R, 1), lambda i: (i, 0)),
            pl.BlockSpec((1, 128), lambda i: (0, 0)),
            pl.BlockSpec((128, 128), lambda i: (0, 0)),
            pl.BlockSpec((_R, 1), lambda i: (i, 0)),
        ],
        out_specs=pl.BlockSpec((_R, 128), lambda i: (i, 0)),
        out_shape=jax.ShapeDtypeStruct((NP, 128), jnp.float32),
    )(parts, ndst, b1r, W2, nsrc)


def _final_body(p_ref, nd_ref, b_ref, w_ref, bp_ref, o_ref):
    x = (p_ref[0] + p_ref[1]) * nd_ref[...] + b_ref[...]
    x = jnp.maximum(x, 0.0)
    lg = jnp.dot(x, w_ref[...], preferred_element_type=jnp.float32) + bp_ref[...]
    m = jnp.max(lg, axis=1, keepdims=True)
    e = jnp.exp(lg - m)
    o_ref[...] = e / jnp.sum(e, axis=1, keepdims=True)


def _final(parts, ndst, b2r, Wpp, bpp):
    return pl.pallas_call(
        _final_body,
        grid=(_G,),
        in_specs=[
            pl.BlockSpec((NC, _R, 128), lambda i: (0, i, 0)),
            pl.BlockSpec((_R, 1), lambda i: (i, 0)),
            pl.BlockSpec((1, 128), lambda i: (0, 0)),
            pl.BlockSpec((128, 128), lambda i: (0, 0)),
            pl.BlockSpec((1, 128), lambda i: (0, 0)),
        ],
        out_specs=pl.BlockSpec((_R, 128), lambda i: (i, 0)),
        out_shape=jax.ShapeDtypeStruct((NP, 128), jnp.float32),
    )(parts, ndst, b2r, Wpp, bpp)


def kernel(features, edge_index, edge_types, W1, b1, W2, b2, Wp, bp):
    L = Wp.shape[1]
    e4d = jnp.stack(
        [edge_index[0].astype(jnp.int32).reshape(NW, NCHUNK, CH),
         edge_index[1].astype(jnp.int32).reshape(NW, NCHUNK, CH)],
        axis=2)

    ei2 = jnp.stack([edge_index[0].astype(jnp.int32).reshape(E // CH, CH),
                     edge_index[1].astype(jnp.int32).reshape(E // CH, CH)])
    deg = _deg_kernel(ei2)[:, :, 0]
    nsrc = lax.rsqrt(jnp.maximum(deg[0], 1.0)).reshape(NP, 1)
    ndst = lax.rsqrt(jnp.maximum(deg[1], 1.0)).reshape(NP, 1)

    fpad = jnp.pad(features, ((0, NP - N), (0, 0)))
    b1r = b1.reshape(1, 128)
    b2r = b2.reshape(1, 128)
    Wpp = jnp.pad(Wp, ((0, 0), (0, 128 - L)))
    bpp = jnp.pad(bp, (0, 128 - L), constant_values=-1e30).reshape(1, 128)

    h1 = _mm1(fpad, W1, nsrc)
    p1 = _agg_kernel(h1, e4d)
    h2 = _mm2(p1, ndst, b1r, W2, nsrc)
    p2 = _agg_kernel(h2, e4d)
    out = _final(p2, ndst, b2r, Wpp, bpp)
    return out[:N, :L]
